# baseline (speedup 1.0000x reference)
import jax
import jax.numpy as jnp
from jax import lax
from jax.experimental import pallas as pl
from jax.experimental.pallas import tpu as pltpu, tpu_sc as plsc

_INFO = plsc.get_sparse_core_info()
_NC, _NS = _INFO.num_cores, _INFO.num_subcores
_NW = _NC * _NS

_B = 4 * 8192
_V = 256
_D = 1024
_DSPLIT = 4
_DT = _D // _DSPLIT
_IG = _NW // _DSPLIT
_PER_G = _B // _IG
_C = 64
_STEPS = _PER_G // _C
_NBUF = 2
_OUTER = _STEPS // _NBUF
_UNROLL = 8


def _body(idx_hbm, table_hbm, out_hbm, idx_v, table_v, rows0, rows1,
          isem, tsem, sem0, sem1):
    rows = (rows0, rows1)
    sems = (sem0, sem1)
    wid = lax.axis_index("s") * _NC + lax.axis_index("c")
    ig = wid // _DSPLIT
    ds = wid % _DSPLIT

    pltpu.make_async_copy(table_hbm.at[ds], table_v, tsem).start()
    pltpu.make_async_copy(idx_hbm.at[pl.ds(ig * _PER_G, _PER_G)], idx_v, isem).start()
    pltpu.make_async_copy(table_hbm.at[ds], table_v, tsem).wait()
    pltpu.make_async_copy(idx_hbm.at[pl.ds(ig * _PER_G, _PER_G)], idx_v, isem).wait()

    lane = lax.iota(jnp.int32, 16)

    def compute(g, b):
        for j16 in range(_C // 16):
            riota = lane + j16 * 16
            idx16 = idx_v[pl.ds(g * _C + j16 * 16, 16)]

            def dloop(dj, dvec):
                for k in range(_UNROLL):
                    val = plsc.load_gather(table_v, [idx16, dvec])
                    plsc.store_scatter(rows[b], [riota, dvec], val)
                    dvec = dvec + 1
                return dvec

            lax.fori_loop(0, _DT // _UNROLL, dloop, lane * 0)

    def emit(g, b):
        return pltpu.make_async_copy(
            rows[b], out_hbm.at[pl.ds(ig * _PER_G + g * _C, _C), ds], sems[b])

    for b in range(_NBUF):
        compute(b, b)
        emit(b, b).start()

    def outer(jj, carry):
        for b in range(_NBUF):
            g = jj * _NBUF + b
            emit(g - _NBUF, b).wait()
            compute(g, b)
            emit(g, b).start()
        return carry

    lax.fori_loop(1, _OUTER, outer, 0)
    for b in range(_NBUF):
        emit((_OUTER - 1) * _NBUF + b, b).wait()


@jax.jit
def _embed_lookup(ids_flat, table_t):
    mesh = plsc.VectorSubcoreMesh(core_axis_name="c", subcore_axis_name="s")
    run = pl.kernel(
        _body,
        out_type=jax.ShapeDtypeStruct((_B, _DSPLIT, _DT), jnp.float32),
        mesh=mesh,
        compiler_params=pltpu.CompilerParams(
            use_tc_tiling_on_sc=False, needs_layout_passes=False),
        scratch_types=[
            pltpu.VMEM((_PER_G,), jnp.int32),
            pltpu.VMEM((_V, _DT), jnp.float32),
            pltpu.VMEM((_C, _DT), jnp.float32),
            pltpu.VMEM((_C, _DT), jnp.float32),
            pltpu.SemaphoreType.DMA,
            pltpu.SemaphoreType.DMA,
            pltpu.SemaphoreType.DMA,
            pltpu.SemaphoreType.DMA,
        ],
    )
    return run(ids_flat, table_t)


def kernel(input_ids, attention_mask, embed):
    ids_flat = input_ids.reshape(-1).astype(jnp.int32)
    table_t = embed.reshape(_V, _DSPLIT, _DT).transpose(1, 0, 2)
    out = _embed_lookup(ids_flat, table_t)
    return out.reshape(input_ids.shape[0], input_ids.shape[1], _D)


# speedup vs baseline: 1.8923x; 1.8923x over previous
import jax
import jax.numpy as jnp
from jax import lax
from jax.experimental import pallas as pl
from jax.experimental.pallas import tpu as pltpu, tpu_sc as plsc

_INFO = plsc.get_sparse_core_info()
_NC, _NS = _INFO.num_cores, _INFO.num_subcores
_NW = _NC * _NS

_B = 4 * 8192
_V = 256
_D = 1024
_DSPLIT = 4
_DT = _D // _DSPLIT
_IG = _NW // _DSPLIT
_PER_G = _B // _IG
_C = 64
_STEPS = _PER_G // _C
_NBUF = 2
_OUTER = _STEPS // _NBUF
_UNROLL = 8


def _body(idx_hbm, table_hbm, out_hbm, idx_v, table_v, rows0, rows1,
          isem, tsem, sem0, sem1):
    rows = (rows0, rows1)
    sems = (sem0, sem1)
    wid = lax.axis_index("s") * _NC + lax.axis_index("c")
    ig = wid // _DSPLIT
    ds = wid % _DSPLIT

    pltpu.make_async_copy(table_hbm.at[ds], table_v, tsem).start()
    pltpu.make_async_copy(idx_hbm.at[pl.ds(ig * _PER_G, _PER_G)], idx_v, isem).start()
    pltpu.make_async_copy(table_hbm.at[ds], table_v, tsem).wait()
    pltpu.make_async_copy(idx_hbm.at[pl.ds(ig * _PER_G, _PER_G)], idx_v, isem).wait()

    lane = lax.iota(jnp.int32, 16)

    def compute(g, b):
        for j16 in range(_C // 16):
            riota = lane + j16 * 16
            idx16 = idx_v[pl.ds(g * _C + j16 * 16, 16)]

            @plsc.parallel_loop(0, _DT, 1, unroll=_UNROLL)
            def _(d):
                dvec = lane * 0 + d
                val = plsc.load_gather(table_v, [idx16, dvec])
                plsc.store_scatter(rows[b], [riota, dvec], val)

    def emit(g, b):
        return pltpu.make_async_copy(
            rows[b], out_hbm.at[pl.ds(ig * _PER_G + g * _C, _C), ds], sems[b])

    for b in range(_NBUF):
        compute(b, b)
        emit(b, b).start()

    def outer(jj, carry):
        for b in range(_NBUF):
            g = jj * _NBUF + b
            emit(g - _NBUF, b).wait()
            compute(g, b)
            emit(g, b).start()
        return carry

    lax.fori_loop(1, _OUTER, outer, 0)
    for b in range(_NBUF):
        emit((_OUTER - 1) * _NBUF + b, b).wait()


@jax.jit
def _embed_lookup(ids_flat, table_t):
    mesh = plsc.VectorSubcoreMesh(core_axis_name="c", subcore_axis_name="s")
    run = pl.kernel(
        _body,
        out_type=jax.ShapeDtypeStruct((_B, _DSPLIT, _DT), jnp.float32),
        mesh=mesh,
        compiler_params=pltpu.CompilerParams(
            use_tc_tiling_on_sc=False, needs_layout_passes=False),
        scratch_types=[
            pltpu.VMEM((_PER_G,), jnp.int32),
            pltpu.VMEM((_V, _DT), jnp.float32),
            pltpu.VMEM((_C, _DT), jnp.float32),
            pltpu.VMEM((_C, _DT), jnp.float32),
            pltpu.SemaphoreType.DMA,
            pltpu.SemaphoreType.DMA,
            pltpu.SemaphoreType.DMA,
            pltpu.SemaphoreType.DMA,
        ],
    )
    return run(ids_flat, table_t)


def kernel(input_ids, attention_mask, embed):
    ids_flat = input_ids.reshape(-1).astype(jnp.int32)
    table_t = embed.reshape(_V, _DSPLIT, _DT).transpose(1, 0, 2)
    out = _embed_lookup(ids_flat, table_t)
    return out.reshape(input_ids.shape[0], input_ids.shape[1], _D)


# speedup vs baseline: 6.1245x; 3.2365x over previous
import jax
import jax.numpy as jnp
from jax import lax
from jax.experimental import pallas as pl
from jax.experimental.pallas import tpu as pltpu, tpu_sc as plsc

_INFO = plsc.get_sparse_core_info()
_NC, _NS = _INFO.num_cores, _INFO.num_subcores
_NW = _NC * _NS

_B = 4 * 8192
_V = 256
_D = 1024
_DSPLIT = 4
_DT = _D // _DSPLIT
_IG = _NW // _DSPLIT
_PER_G = _B // _IG
_C = 64
_STEPS = _PER_G // _C
_NBUF = 2
_OUTER = _STEPS // _NBUF
_UNROLL = 8


def _body(idx_hbm, table_hbm, out_hbm, idx_v, table_v, rows0, rows1,
          isem, tsem, sem0, sem1):
    rows = (rows0, rows1)
    sems = (sem0, sem1)
    wid = lax.axis_index("s") * _NC + lax.axis_index("c")
    ig = wid // _DSPLIT
    ds = wid % _DSPLIT

    pltpu.make_async_copy(table_hbm.at[ds], table_v, tsem).start()
    pltpu.make_async_copy(idx_hbm.at[pl.ds(ig * _PER_G, _PER_G)], idx_v, isem).start()
    pltpu.make_async_copy(table_hbm.at[ds], table_v, tsem).wait()
    pltpu.make_async_copy(idx_hbm.at[pl.ds(ig * _PER_G, _PER_G)], idx_v, isem).wait()

    lane = lax.iota(jnp.int32, 16)

    def compute(g, b):
        for j16 in range(_C // 16):
            riota = lane + j16 * 16
            idx16 = idx_v[pl.ds(g * _C + j16 * 16, 16)]

            @plsc.parallel_loop(0, _DT, 1, unroll=_UNROLL)
            def _(d):
                dvec = (lane + d) & (_DT - 1)
                val = plsc.load_gather(table_v, [idx16, dvec])
                plsc.store_scatter(rows[b], [riota, dvec], val)

    def emit(g, b):
        return pltpu.make_async_copy(
            rows[b], out_hbm.at[pl.ds(ig * _PER_G + g * _C, _C), ds], sems[b])

    for b in range(_NBUF):
        compute(b, b)
        emit(b, b).start()

    def outer(jj, carry):
        for b in range(_NBUF):
            g = jj * _NBUF + b
            emit(g - _NBUF, b).wait()
            compute(g, b)
            emit(g, b).start()
        return carry

    lax.fori_loop(1, _OUTER, outer, 0)
    for b in range(_NBUF):
        emit((_OUTER - 1) * _NBUF + b, b).wait()


@jax.jit
def _embed_lookup(ids_flat, table_t):
    mesh = plsc.VectorSubcoreMesh(core_axis_name="c", subcore_axis_name="s")
    run = pl.kernel(
        _body,
        out_type=jax.ShapeDtypeStruct((_B, _DSPLIT, _DT), jnp.float32),
        mesh=mesh,
        compiler_params=pltpu.CompilerParams(
            use_tc_tiling_on_sc=False, needs_layout_passes=False),
        scratch_types=[
            pltpu.VMEM((_PER_G,), jnp.int32),
            pltpu.VMEM((_V, _DT), jnp.float32),
            pltpu.VMEM((_C, _DT), jnp.float32),
            pltpu.VMEM((_C, _DT), jnp.float32),
            pltpu.SemaphoreType.DMA,
            pltpu.SemaphoreType.DMA,
            pltpu.SemaphoreType.DMA,
            pltpu.SemaphoreType.DMA,
        ],
    )
    return run(ids_flat, table_t)


def kernel(input_ids, attention_mask, embed):
    ids_flat = input_ids.reshape(-1).astype(jnp.int32)
    table_t = embed.reshape(_V, _DSPLIT, _DT).transpose(1, 0, 2)
    out = _embed_lookup(ids_flat, table_t)
    return out.reshape(input_ids.shape[0], input_ids.shape[1], _D)


# speedup vs baseline: 6.6558x; 1.0867x over previous
import jax
import jax.numpy as jnp
from jax import lax
from jax.experimental import pallas as pl
from jax.experimental.pallas import tpu as pltpu, tpu_sc as plsc

_INFO = plsc.get_sparse_core_info()
_NC, _NS = _INFO.num_cores, _INFO.num_subcores
_NW = _NC * _NS

_B = 4 * 8192
_V = 256
_D = 1024
_DSPLIT = 4
_DT = _D // _DSPLIT
_IG = _NW // _DSPLIT
_PER_G = _B // _IG
_C = 16
_STEPS = _PER_G // _C


def _body(idx_hbm, table_hbm, out_hbm, idx_v, table_v, drain_v, isem, tsem, sem):
    wid = lax.axis_index("s") * _NC + lax.axis_index("c")
    ig = wid // _DSPLIT
    ds = wid % _DSPLIT
    gbase = ig * _PER_G

    pltpu.make_async_copy(table_hbm.at[ds], table_v, tsem).start()
    pltpu.make_async_copy(idx_hbm.at[pl.ds(gbase, _PER_G)], idx_v, isem).start()
    pltpu.make_async_copy(table_hbm.at[ds], table_v, tsem).wait()
    pltpu.make_async_copy(idx_hbm.at[pl.ds(gbase, _PER_G)], idx_v, isem).wait()

    def fire(g):
        ids16 = idx_v[pl.ds(g * _C, _C)]
        for j in range(_C):
            pltpu.make_async_copy(
                table_v.at[ids16[j]], out_hbm.at[gbase + g * _C + j, ds], sem).start()

    def drain():
        pltpu.make_async_copy(out_hbm.at[pl.ds(gbase, _C), ds], drain_v, sem).wait()

    fire(0)

    def chunk(g, carry):
        fire(g)
        drain()
        return carry

    lax.fori_loop(1, _STEPS, chunk, 0)
    drain()


@jax.jit
def _embed_lookup(ids_flat, table_t):
    mesh = plsc.VectorSubcoreMesh(core_axis_name="c", subcore_axis_name="s")
    run = pl.kernel(
        _body,
        out_type=jax.ShapeDtypeStruct((_B, _DSPLIT, _DT), jnp.float32),
        mesh=mesh,
        compiler_params=pltpu.CompilerParams(
            use_tc_tiling_on_sc=False, needs_layout_passes=False),
        scratch_types=[
            pltpu.VMEM((_PER_G,), jnp.int32),
            pltpu.VMEM((_V, _DT), jnp.float32),
            pltpu.VMEM((_C, _DT), jnp.float32),
            pltpu.SemaphoreType.DMA,
            pltpu.SemaphoreType.DMA,
            pltpu.SemaphoreType.DMA,
        ],
    )
    return run(ids_flat, table_t)


def kernel(input_ids, attention_mask, embed):
    ids_flat = input_ids.reshape(-1).astype(jnp.int32)
    table_t = embed.reshape(_V, _DSPLIT, _DT).transpose(1, 0, 2)
    out = _embed_lookup(ids_flat, table_t)
    return out.reshape(input_ids.shape[0], input_ids.shape[1], _D)
